# baseline (speedup 1.0000x reference)
import functools

import jax
import jax.numpy as jnp
from jax import lax
from jax.experimental import pallas as pl
from jax.experimental.pallas import tpu as pltpu
from jax.experimental.pallas import tpu_sc as plsc

_N = 10000
_F = 128
_NC = 2
_NS = 16
_NW = _NC * _NS
_CHUNK = 128
_NPAD = 10240
_RPT = _NPAD // _NS
_UNROLL = 2


def _mesh():
    return plsc.VectorSubcoreMesh(
        core_axis_name="c", subcore_axis_name="s",
        num_cores=_NC, num_subcores=_NS)


def _sc_scatter(table, edges_p, zeros2d):
    nit = edges_p[0].shape[0] // (_NW * _CHUNK)
    assert nit % _UNROLL == 0

    def body(table_h, src_h, dst_h, zeros_h, out_h,
             src0, src1, dst0, dst1, rows0, rows1,
             gsem0, gsem1, ssem, acc_s):
        srcs = (src0, src1)
        dsts = (dst0, dst1)
        rows = (rows0, rows1)
        gsems = (gsem0, gsem1)
        cid = lax.axis_index("c")
        sid = lax.axis_index("s")
        r0 = sid * _RPT
        pltpu.sync_copy(zeros_h.at[pl.ds(r0, _RPT), :], acc_s.at[pl.ds(r0, _RPT), :])
        plsc.subcore_barrier()
        ebase = (cid * _NS + sid) * nit * _CHUNK

        def it(i, carry):
            base = ebase + i * (_UNROLL * _CHUNK)
            gds = []
            for k in range(_UNROLL):
                b = pl.multiple_of(base + k * _CHUNK, 8)
                pltpu.sync_copy(src_h.at[pl.ds(b, _CHUNK)], srcs[k])
                pltpu.sync_copy(dst_h.at[pl.ds(b, _CHUNK)], dsts[k])
                gds.append(pltpu.async_copy(table_h.at[srcs[k]], rows[k], gsems[k]))
            sds = []
            for k in range(_UNROLL):
                gds[k].wait()
                sds.append(pltpu.async_copy(rows[k], acc_s.at[dsts[k]], ssem,
                                            add=True))
            for k in range(_UNROLL):
                sds[k].wait()
            return carry

        lax.fori_loop(0, nit // _UNROLL, it, 0)
        plsc.subcore_barrier()
        pltpu.sync_copy(acc_s.at[pl.ds(r0, _RPT), :],
                        out_h.at[cid, pl.ds(r0, _RPT), :])

    return pl.kernel(
        body,
        out_type=jax.ShapeDtypeStruct((_NC, _NPAD, _F), jnp.float32),
        mesh=_mesh(),
        scratch_types=(
            [pltpu.VMEM((_CHUNK,), jnp.int32)] * 4
            + [pltpu.VMEM((_CHUNK, _F), jnp.float32)] * 2
            + [pltpu.SemaphoreType.DMA] * 3
            + [pltpu.VMEM_SHARED((_NPAD, _F), jnp.float32)]
        ),
    )(table, edges_p[0], edges_p[1], zeros2d)


def _sc_degree(edges_p, zeros1d):
    nit = edges_p[1].shape[0] // (_NW * _CHUNK)

    def body(dst_h, zeros_h, out_h, dst_v, ones_v, acc_s):
        cid = lax.axis_index("c")
        sid = lax.axis_index("s")
        r0 = sid * _RPT
        pltpu.sync_copy(zeros_h.at[pl.ds(r0, _RPT)], acc_s.at[pl.ds(r0, _RPT)])
        for j in range(_CHUNK // 16):
            ones_v[pl.ds(j * 16, 16)] = jnp.ones((16,), jnp.float32)
        plsc.subcore_barrier()
        ebase = (cid * _NS + sid) * nit * _CHUNK

        def it(i, carry):
            b = pl.multiple_of(ebase + i * _CHUNK, 8)
            pltpu.sync_copy(dst_h.at[pl.ds(b, _CHUNK)], dst_v)
            pltpu.sync_copy(ones_v, acc_s.at[dst_v], add=True)
            return carry

        lax.fori_loop(0, nit, it, 0)
        plsc.subcore_barrier()
        pltpu.sync_copy(acc_s.at[pl.ds(r0, _RPT)], out_h.at[cid, pl.ds(r0, _RPT)])

    return pl.kernel(
        body,
        out_type=jax.ShapeDtypeStruct((_NC, _NPAD), jnp.float32),
        mesh=_mesh(),
        scratch_types=[
            pltpu.VMEM((_CHUNK,), jnp.int32),
            pltpu.VMEM((_CHUNK,), jnp.float32),
            pltpu.VMEM_SHARED((_NPAD,), jnp.float32),
        ],
    )(edges_p[1], zeros1d)


_R = 1000


def _dinv_of(degp_blk):
    deg = degp_blk[:, 0] + degp_blk[:, 1] + 1.0
    return lax.rsqrt(deg)[:, None]


def _tc_prescale(degp_t, x):
    def body(degp_ref, x_ref, o_ref):
        o_ref[...] = _dinv_of(degp_ref[...]) * x_ref[...]

    return pl.pallas_call(
        body,
        grid=(_N // _R,),
        in_specs=[
            pl.BlockSpec((_R, _NC), lambda i: (i, 0)),
            pl.BlockSpec((_R, _F), lambda i: (i, 0)),
        ],
        out_specs=pl.BlockSpec((_R, _F), lambda i: (i, 0)),
        out_shape=jax.ShapeDtypeStruct((_N, _F), jnp.float32),
    )(degp_t, x)


def _tc_layer1(degp_t, s1, xs, W1, b1):
    def body(degp_ref, s1_ref, xs_ref, w_ref, b_ref, oa_ref, ob_ref):
        dinv = _dinv_of(degp_ref[...])
        u = dinv * (s1_ref[0] + s1_ref[1] + xs_ref[...])
        h = jnp.dot(u, w_ref[...], preferred_element_type=jnp.float32) + b_ref[...]
        hs = dinv * jnp.maximum(h, 0.0)
        oa_ref[...] = hs[:, :_F]
        ob_ref[...] = hs[:, _F:]

    return pl.pallas_call(
        body,
        grid=(_N // _R,),
        in_specs=[
            pl.BlockSpec((_R, _NC), lambda i: (i, 0)),
            pl.BlockSpec((_NC, _R, _F), lambda i: (0, i, 0)),
            pl.BlockSpec((_R, _F), lambda i: (i, 0)),
            pl.BlockSpec((_F, 2 * _F), lambda i: (0, 0)),
            pl.BlockSpec((1, 2 * _F), lambda i: (0, 0)),
        ],
        out_specs=[
            pl.BlockSpec((_R, _F), lambda i: (i, 0)),
            pl.BlockSpec((_R, _F), lambda i: (i, 0)),
        ],
        out_shape=[
            jax.ShapeDtypeStruct((_N, _F), jnp.float32),
            jax.ShapeDtypeStruct((_N, _F), jnp.float32),
        ],
    )(degp_t, s1, xs, W1, b1)


def _tc_layer2(degp_t, s2a, s2b, h1a, h1b, W2, b2, W3):
    def body(degp_ref, sa_ref, sb_ref, ha_ref, hb_ref, w2_ref, b2_ref, w3_ref, o_ref):
        dinv = _dinv_of(degp_ref[...])
        ua = dinv * (sa_ref[0] + sa_ref[1] + ha_ref[...])
        ub = dinv * (sb_ref[0] + sb_ref[1] + hb_ref[...])
        h2 = (jnp.dot(ua, w2_ref[:_F, :], preferred_element_type=jnp.float32)
              + jnp.dot(ub, w2_ref[_F:, :], preferred_element_type=jnp.float32)
              + b2_ref[...])
        h2 = jnp.maximum(h2, 0.0)
        g = jnp.dot(h2, w3_ref[...], preferred_element_type=jnp.float32)
        o_ref[...] = dinv * g

    return pl.pallas_call(
        body,
        grid=(_N // _R,),
        in_specs=[
            pl.BlockSpec((_R, _NC), lambda i: (i, 0)),
            pl.BlockSpec((_NC, _R, _F), lambda i: (0, i, 0)),
            pl.BlockSpec((_NC, _R, _F), lambda i: (0, i, 0)),
            pl.BlockSpec((_R, _F), lambda i: (i, 0)),
            pl.BlockSpec((_R, _F), lambda i: (i, 0)),
            pl.BlockSpec((2 * _F, 2 * _F), lambda i: (0, 0)),
            pl.BlockSpec((1, 2 * _F), lambda i: (0, 0)),
            pl.BlockSpec((2 * _F, _F), lambda i: (0, 0)),
        ],
        out_specs=pl.BlockSpec((_R, _F), lambda i: (i, 0)),
        out_shape=jax.ShapeDtypeStruct((_N, _F), jnp.float32),
    )(degp_t, s2a, s2b, h1a, h1b, W2, b2, W3)


def _tc_layer3(degp_t, s3, gs, b3):
    def body(degp_ref, s3_ref, gs_ref, b_ref, o_ref):
        dinv = _dinv_of(degp_ref[...])
        o_ref[...] = dinv * (s3_ref[0] + s3_ref[1] + gs_ref[...]) + b_ref[...]

    return pl.pallas_call(
        body,
        grid=(_N // _R,),
        in_specs=[
            pl.BlockSpec((_R, _NC), lambda i: (i, 0)),
            pl.BlockSpec((_NC, _R, _F), lambda i: (0, i, 0)),
            pl.BlockSpec((_R, _F), lambda i: (i, 0)),
            pl.BlockSpec((1, _F), lambda i: (0, 0)),
        ],
        out_specs=pl.BlockSpec((_R, _F), lambda i: (i, 0)),
        out_shape=jax.ShapeDtypeStruct((_N, _F), jnp.float32),
    )(degp_t, s3, gs, b3)


def kernel(x, edge_index, W1, b1, W2, b2, W3, b3):
    E = edge_index.shape[1]
    grain = _NW * _CHUNK * _UNROLL
    epad = ((E + grain - 1) // grain) * grain
    src = edge_index[0]
    dst = edge_index[1]
    if epad != E:
        pad = epad - E
        src = jnp.concatenate([src, jnp.zeros((pad,), jnp.int32)])
        dst = jnp.concatenate([dst, jnp.full((pad,), _N, jnp.int32)])
    edges_p = (src, dst)
    zeros2d = jnp.zeros((_NPAD, _F), jnp.float32)
    zeros1d = jnp.zeros((_NPAD,), jnp.float32)

    degp = _sc_degree(edges_p, zeros1d)
    degp_t = degp[:, :_N].T
    xs = _tc_prescale(degp_t, x)
    s1 = _sc_scatter(xs, edges_p, zeros2d)
    h1a, h1b = _tc_layer1(degp_t, s1, xs, W1, b1.reshape(1, -1))
    s2a = _sc_scatter(h1a, edges_p, zeros2d)
    s2b = _sc_scatter(h1b, edges_p, zeros2d)
    gs = _tc_layer2(degp_t, s2a, s2b, h1a, h1b, W2, b2.reshape(1, -1), W3)
    s3 = _sc_scatter(gs, edges_p, zeros2d)
    return _tc_layer3(degp_t, s3, gs, b3.reshape(1, -1))


# speedup vs baseline: 1.4632x; 1.4632x over previous
import functools

import jax
import jax.numpy as jnp
from jax import lax
from jax.experimental import pallas as pl
from jax.experimental.pallas import tpu as pltpu
from jax.experimental.pallas import tpu_sc as plsc

_N = 10000
_F = 128
_NC = 2
_NS = 16
_NW = _NC * _NS
_CHUNK = 128
_NPAD = 10240
_RPT = _NPAD // _NS


def _mesh():
    return plsc.VectorSubcoreMesh(
        core_axis_name="c", subcore_axis_name="s",
        num_cores=_NC, num_subcores=_NS)


def _sc_scatter(table, edges_p, zeros2d):
    nit = edges_p[1].shape[0] // (_NW * _CHUNK)

    def body(table_h, idx2_h, zeros_h, out_h, idx_v, rows_v, gsem, acc_s):
        cid = lax.axis_index("c")
        sid = lax.axis_index("s")
        r0 = sid * _RPT
        pltpu.sync_copy(zeros_h.at[pl.ds(r0, _RPT), :], acc_s.at[pl.ds(r0, _RPT), :])
        plsc.subcore_barrier()
        cbase = (cid * _NS + sid) * nit

        def it(i, carry):
            pltpu.sync_copy(idx2_h.at[pl.ds(2 * (cbase + i), 2), :], idx_v)
            pltpu.async_copy(table_h.at[idx_v.at[0]], rows_v, gsem).wait()
            pltpu.sync_copy(rows_v, acc_s.at[idx_v.at[1]], add=True)
            return carry

        lax.fori_loop(0, nit, it, 0)
        plsc.subcore_barrier()
        pltpu.sync_copy(acc_s.at[pl.ds(r0, _RPT), :],
                        out_h.at[cid, pl.ds(r0, _RPT), :])

    return pl.kernel(
        body,
        out_type=jax.ShapeDtypeStruct((_NC, _NPAD, _F), jnp.float32),
        mesh=_mesh(),
        scratch_types=[
            pltpu.VMEM((2, _CHUNK), jnp.int32),
            pltpu.VMEM((_CHUNK, _F), jnp.float32),
            pltpu.SemaphoreType.DMA,
            pltpu.VMEM_SHARED((_NPAD, _F), jnp.float32),
        ],
    )(table, edges_p[0], zeros2d)


def _sc_degree(edges_p, zeros1d):
    nit = edges_p[1].shape[0] // (_NW * _CHUNK)

    def body(dst_h, zeros_h, out_h, dst_v, ones_v, acc_s):
        cid = lax.axis_index("c")
        sid = lax.axis_index("s")
        r0 = sid * _RPT
        pltpu.sync_copy(zeros_h.at[pl.ds(r0, _RPT)], acc_s.at[pl.ds(r0, _RPT)])
        for j in range(_CHUNK // 16):
            ones_v[pl.ds(j * 16, 16)] = jnp.ones((16,), jnp.float32)
        plsc.subcore_barrier()
        ebase = (cid * _NS + sid) * nit * _CHUNK

        def it(i, carry):
            b = pl.multiple_of(ebase + i * _CHUNK, 8)
            pltpu.sync_copy(dst_h.at[pl.ds(b, _CHUNK)], dst_v)
            pltpu.sync_copy(ones_v, acc_s.at[dst_v], add=True)
            return carry

        lax.fori_loop(0, nit, it, 0)
        plsc.subcore_barrier()
        pltpu.sync_copy(acc_s.at[pl.ds(r0, _RPT)], out_h.at[cid, pl.ds(r0, _RPT)])

    return pl.kernel(
        body,
        out_type=jax.ShapeDtypeStruct((_NC, _NPAD), jnp.float32),
        mesh=_mesh(),
        scratch_types=[
            pltpu.VMEM((_CHUNK,), jnp.int32),
            pltpu.VMEM((_CHUNK,), jnp.float32),
            pltpu.VMEM_SHARED((_NPAD,), jnp.float32),
        ],
    )(edges_p[1], zeros1d)


_R = 1000


def _dinv_of(degp_blk):
    deg = degp_blk[:, 0] + degp_blk[:, 1] + 1.0
    return lax.rsqrt(deg)[:, None]


def _tc_prescale(degp_t, x):
    def body(degp_ref, x_ref, o_ref):
        o_ref[...] = _dinv_of(degp_ref[...]) * x_ref[...]

    return pl.pallas_call(
        body,
        grid=(_N // _R,),
        in_specs=[
            pl.BlockSpec((_R, _NC), lambda i: (i, 0)),
            pl.BlockSpec((_R, _F), lambda i: (i, 0)),
        ],
        out_specs=pl.BlockSpec((_R, _F), lambda i: (i, 0)),
        out_shape=jax.ShapeDtypeStruct((_N, _F), jnp.float32),
    )(degp_t, x)


def _tc_layer1(degp_t, s1, xs, W1, b1):
    def body(degp_ref, s1_ref, xs_ref, w_ref, b_ref, oa_ref, ob_ref):
        dinv = _dinv_of(degp_ref[...])
        u = dinv * (s1_ref[0] + s1_ref[1] + xs_ref[...])
        h = jnp.dot(u, w_ref[...], preferred_element_type=jnp.float32) + b_ref[...]
        hs = dinv * jnp.maximum(h, 0.0)
        oa_ref[...] = hs[:, :_F]
        ob_ref[...] = hs[:, _F:]

    return pl.pallas_call(
        body,
        grid=(_N // _R,),
        in_specs=[
            pl.BlockSpec((_R, _NC), lambda i: (i, 0)),
            pl.BlockSpec((_NC, _R, _F), lambda i: (0, i, 0)),
            pl.BlockSpec((_R, _F), lambda i: (i, 0)),
            pl.BlockSpec((_F, 2 * _F), lambda i: (0, 0)),
            pl.BlockSpec((1, 2 * _F), lambda i: (0, 0)),
        ],
        out_specs=[
            pl.BlockSpec((_R, _F), lambda i: (i, 0)),
            pl.BlockSpec((_R, _F), lambda i: (i, 0)),
        ],
        out_shape=[
            jax.ShapeDtypeStruct((_N, _F), jnp.float32),
            jax.ShapeDtypeStruct((_N, _F), jnp.float32),
        ],
    )(degp_t, s1, xs, W1, b1)


def _tc_layer2(degp_t, s2a, s2b, h1a, h1b, W2, b2, W3):
    def body(degp_ref, sa_ref, sb_ref, ha_ref, hb_ref, w2_ref, b2_ref, w3_ref, o_ref):
        dinv = _dinv_of(degp_ref[...])
        ua = dinv * (sa_ref[0] + sa_ref[1] + ha_ref[...])
        ub = dinv * (sb_ref[0] + sb_ref[1] + hb_ref[...])
        h2 = (jnp.dot(ua, w2_ref[:_F, :], preferred_element_type=jnp.float32)
              + jnp.dot(ub, w2_ref[_F:, :], preferred_element_type=jnp.float32)
              + b2_ref[...])
        h2 = jnp.maximum(h2, 0.0)
        g = jnp.dot(h2, w3_ref[...], preferred_element_type=jnp.float32)
        o_ref[...] = dinv * g

    return pl.pallas_call(
        body,
        grid=(_N // _R,),
        in_specs=[
            pl.BlockSpec((_R, _NC), lambda i: (i, 0)),
            pl.BlockSpec((_NC, _R, _F), lambda i: (0, i, 0)),
            pl.BlockSpec((_NC, _R, _F), lambda i: (0, i, 0)),
            pl.BlockSpec((_R, _F), lambda i: (i, 0)),
            pl.BlockSpec((_R, _F), lambda i: (i, 0)),
            pl.BlockSpec((2 * _F, 2 * _F), lambda i: (0, 0)),
            pl.BlockSpec((1, 2 * _F), lambda i: (0, 0)),
            pl.BlockSpec((2 * _F, _F), lambda i: (0, 0)),
        ],
        out_specs=pl.BlockSpec((_R, _F), lambda i: (i, 0)),
        out_shape=jax.ShapeDtypeStruct((_N, _F), jnp.float32),
    )(degp_t, s2a, s2b, h1a, h1b, W2, b2, W3)


def _tc_layer3(degp_t, s3, gs, b3):
    def body(degp_ref, s3_ref, gs_ref, b_ref, o_ref):
        dinv = _dinv_of(degp_ref[...])
        o_ref[...] = dinv * (s3_ref[0] + s3_ref[1] + gs_ref[...]) + b_ref[...]

    return pl.pallas_call(
        body,
        grid=(_N // _R,),
        in_specs=[
            pl.BlockSpec((_R, _NC), lambda i: (i, 0)),
            pl.BlockSpec((_NC, _R, _F), lambda i: (0, i, 0)),
            pl.BlockSpec((_R, _F), lambda i: (i, 0)),
            pl.BlockSpec((1, _F), lambda i: (0, 0)),
        ],
        out_specs=pl.BlockSpec((_R, _F), lambda i: (i, 0)),
        out_shape=jax.ShapeDtypeStruct((_N, _F), jnp.float32),
    )(degp_t, s3, gs, b3)


def kernel(x, edge_index, W1, b1, W2, b2, W3, b3):
    E = edge_index.shape[1]
    grain = _NW * _CHUNK
    epad = ((E + grain - 1) // grain) * grain
    src = edge_index[0]
    dst = edge_index[1]
    if epad != E:
        pad = epad - E
        src = jnp.concatenate([src, jnp.zeros((pad,), jnp.int32)])
        dst = jnp.concatenate([dst, jnp.full((pad,), _N, jnp.int32)])
    nch = epad // _CHUNK
    idx2 = jnp.stack([src.reshape(nch, _CHUNK), dst.reshape(nch, _CHUNK)],
                     axis=1).reshape(2 * nch, _CHUNK)
    edges_p = (idx2, dst)
    zeros2d = jnp.zeros((_NPAD, _F), jnp.float32)
    zeros1d = jnp.zeros((_NPAD,), jnp.float32)

    degp = _sc_degree(edges_p, zeros1d)
    degp_t = degp[:, :_N].T
    xs = _tc_prescale(degp_t, x)
    s1 = _sc_scatter(xs, edges_p, zeros2d)
    h1a, h1b = _tc_layer1(degp_t, s1, xs, W1, b1.reshape(1, -1))
    s2a = _sc_scatter(h1a, edges_p, zeros2d)
    s2b = _sc_scatter(h1b, edges_p, zeros2d)
    gs = _tc_layer2(degp_t, s2a, s2b, h1a, h1b, W2, b2.reshape(1, -1), W3)
    s3 = _sc_scatter(gs, edges_p, zeros2d)
    return _tc_layer3(degp_t, s3, gs, b3.reshape(1, -1))
